# Initial kernel scaffold; baseline (speedup 1.0000x reference)
import jax
import jax.numpy as jnp
from jax.experimental import pallas as pl


def kernel(x, Wq, Wk, Wv, Wu, bu):
    raise NotImplementedError("write your pallas kernel here")


# speedup vs baseline: 41.0165x; 41.0165x over previous
import jax
import jax.numpy as jnp
from jax.experimental import pallas as pl

_E = 128
_H = 8
_K = 32
_BT = 256


def _band_attn_kernel(x_ref, xp_ref, wq_ref, wk_ref, wv_ref, wu_ref,
                      bu_ref, out_ref):
    e, h, k, bt = _E, _H, _K, _BT
    t = x_ref.shape[0]
    nb = t // bt
    scale = jnp.float32(1.0 / (e ** 0.5))

    x = x_ref[...]
    xp = xp_ref[...]
    q = jnp.dot(x, wq_ref[...], preferred_element_type=jnp.float32) * scale
    kk = jnp.dot(xp, wk_ref[...], preferred_element_type=jnp.float32)
    vv = jnp.dot(xp, wv_ref[...], preferred_element_type=jnp.float32)
    wu = wu_ref[...]
    bu = bu_ref[...]

    rows = jax.lax.broadcasted_iota(jnp.int32, (bt, bt + k), 0)
    cols = jax.lax.broadcasted_iota(jnp.int32, (bt, bt + k), 1)
    band = (cols >= rows) & (cols <= rows + (k - 1))

    for i in range(nb):
        qb = q[i * bt:(i + 1) * bt, :]
        kb = kk[i * bt:i * bt + bt + k, :]
        vb = vv[i * bt:i * bt + bt + k, :]
        acc = jnp.broadcast_to(bu, (bt, e)).astype(jnp.float32)
        for hh in range(h):
            qh = qb[:, hh * e:(hh + 1) * e]
            kh = kb[:, hh * e:(hh + 1) * e]
            vh = vb[:, hh * e:(hh + 1) * e]
            s = jax.lax.dot_general(qh, kh, (((1,), (1,)), ((), ())),
                                    preferred_element_type=jnp.float32)
            s = jnp.where(band, s, jnp.float32(-1e30))
            m = jnp.max(s, axis=1, keepdims=True)
            p = jnp.exp(s - m)
            att = p / jnp.sum(p, axis=1, keepdims=True)
            oh = jnp.dot(att, vh, preferred_element_type=jnp.float32)
            acc = acc + jnp.dot(oh, wu[hh * e:(hh + 1) * e, :],
                                preferred_element_type=jnp.float32)
        out_ref[i * bt:(i + 1) * bt, :] = acc


def kernel(x, Wq, Wk, Wv, Wu, bu):
    b, t, e = x.shape
    x2 = x[0]
    xp = jnp.pad(x2, ((_K - 1, 1), (0, 0)))
    bu2 = bu.reshape(1, e)
    out = pl.pallas_call(
        _band_attn_kernel,
        out_shape=jax.ShapeDtypeStruct((t, e), jnp.float32),
    )(x2, xp, Wq, Wk, Wv, Wu, bu2)
    return out[None]


# speedup vs baseline: 56.7838x; 1.3844x over previous
import jax
import jax.numpy as jnp
from jax.experimental import pallas as pl

_E = 128
_H = 8
_K = 32
_BT = 256


def _band_attn_kernel(x_ref, xp_ref, wq_ref, wk_ref, wv_ref, wu_ref,
                      bu_ref, out_ref):
    e, h, k, bt = _E, _H, _K, _BT
    t = x_ref.shape[0]
    nb = t // bt
    scale = jnp.float32(1.0 / (e ** 0.5))

    x = x_ref[...]
    xp = xp_ref[...]
    q = jnp.dot(x, wq_ref[...], preferred_element_type=jnp.float32) * scale
    kk = jnp.dot(xp, wk_ref[...], preferred_element_type=jnp.float32)
    vv = jnp.dot(xp, wv_ref[...], preferred_element_type=jnp.float32)
    wu = wu_ref[...]
    bu = bu_ref[...]

    rows = jax.lax.broadcasted_iota(jnp.int32, (bt, bt + k), 0)
    cols = jax.lax.broadcasted_iota(jnp.int32, (bt, bt + k), 1)
    band = (cols >= rows) & (cols <= rows + (k - 1))

    for i in range(nb):
        qb = q[i * bt:(i + 1) * bt, :]
        kb = kk[i * bt:i * bt + bt + k, :]
        vb = vv[i * bt:i * bt + bt + k, :]
        heads = []
        for hh in range(h):
            qh = qb[:, hh * e:(hh + 1) * e]
            kh = kb[:, hh * e:(hh + 1) * e]
            vh = vb[:, hh * e:(hh + 1) * e]
            s = jax.lax.dot_general(qh, kh, (((1,), (1,)), ((), ())),
                                    preferred_element_type=jnp.float32)
            s = jnp.where(band, s, jnp.float32(-1e30))
            m = jnp.max(s, axis=1, keepdims=True)
            p = jnp.exp(s - m)
            r = jnp.float32(1.0) / jnp.sum(p, axis=1, keepdims=True)
            oh = jnp.dot(p, vh, preferred_element_type=jnp.float32) * r
            heads.append(oh)
        hcat = jnp.concatenate(heads, axis=1)
        acc = jnp.dot(hcat, wu, preferred_element_type=jnp.float32) + bu
        out_ref[i * bt:(i + 1) * bt, :] = acc


def kernel(x, Wq, Wk, Wv, Wu, bu):
    b, t, e = x.shape
    x2 = x[0]
    xp = jnp.pad(x2, ((_K - 1, 1), (0, 0)))
    bu2 = bu.reshape(1, e)
    out = pl.pallas_call(
        _band_attn_kernel,
        out_shape=jax.ShapeDtypeStruct((t, e), jnp.float32),
    )(x2, xp, Wq, Wk, Wv, Wu, bu2)
    return out[None]
